# Initial kernel scaffold; baseline (speedup 1.0000x reference)
import jax
import jax.numpy as jnp
from jax.experimental import pallas as pl


def kernel(cls_logits, bbox_regression, anchors):
    raise NotImplementedError("write your pallas kernel here")


# speedup vs baseline: 2.5920x; 2.5920x over previous
import math

import jax
import jax.numpy as jnp
from jax.experimental import pallas as pl

_N = 25000
_C = 81
_K = 1000
_KP = 1024
_MAXD = 200
_H = 512.0
_W = 512.0
_SCORE_THR = 0.01
_NMS_THR = 0.45
_CLIP = math.log(1000.0 / 16.0)

_NP = 25600
_BR = 1024
_GRID = _NP // _BR


def _prep_kernel(logits_ref, reg_ref, anc_ref, scores_ref, boxes_ref):
    logits = logits_ref[...]
    m = jnp.max(logits, axis=-1, keepdims=True)
    e = jnp.exp(logits - m)
    probs = e / jnp.sum(e, axis=-1, keepdims=True)
    sc = probs[:, 1:]
    scores_ref[...] = jnp.where(sc >= _SCORE_THR, sc, jnp.zeros_like(sc))

    anc = anc_ref[...]
    rel = reg_ref[...]
    ax1 = anc[0:1, :]
    ay1 = anc[1:2, :]
    ax2 = anc[2:3, :]
    ay2 = anc[3:4, :]
    w = ax2 - ax1
    h = ay2 - ay1
    cx = ax1 + 0.5 * w
    cy = ay1 + 0.5 * h
    dx = rel[0:1, :] / 10.0
    dy = rel[1:2, :] / 10.0
    dw = jnp.minimum(rel[2:3, :] / 5.0, _CLIP)
    dh = jnp.minimum(rel[3:4, :] / 5.0, _CLIP)
    pcx = dx * w + cx
    pcy = dy * h + cy
    pw = jnp.exp(dw) * w
    ph = jnp.exp(dh) * h
    bx1 = jnp.clip(pcx - 0.5 * pw, 0.0, _W)
    by1 = jnp.clip(pcy - 0.5 * ph, 0.0, _H)
    bx2 = jnp.clip(pcx + 0.5 * pw, 0.0, _W)
    by2 = jnp.clip(pcy + 0.5 * ph, 0.0, _H)
    boxes_ref[...] = jnp.concatenate([bx1, by1, bx2, by2], axis=0)


def _nms_kernel(b_ref, s_ref, l_ref, keep_ref):
    x1 = b_ref[0:1, :]
    y1 = b_ref[1:2, :]
    x2 = b_ref[2:3, :]
    y2 = b_ref[3:4, :]
    scores = s_ref[...]
    labels = l_ref[...]
    areas = jnp.maximum(x2 - x1, 0.0) * jnp.maximum(y2 - y1, 0.0)
    iota = jax.lax.broadcasted_iota(jnp.int32, (1, _KP), 1)

    def body(i, carry):
        suppressed, keep = carry
        onehot = jnp.where(iota == i, 1.0, 0.0)

        def pick(v):
            return jnp.sum(v * onehot, axis=-1, keepdims=True)

        xi1 = pick(x1)
        yi1 = pick(y1)
        xi2 = pick(x2)
        yi2 = pick(y2)
        ai = pick(areas)
        si = pick(scores)
        li = pick(labels)
        supi = pick(suppressed)

        xx1 = jnp.maximum(x1, xi1)
        yy1 = jnp.maximum(y1, yi1)
        xx2 = jnp.minimum(x2, xi2)
        yy2 = jnp.minimum(y2, yi2)
        inter = jnp.maximum(xx2 - xx1, 0.0) * jnp.maximum(yy2 - yy1, 0.0)
        union = areas + ai - inter
        iou = jnp.where(union > 0.0, inter / union, jnp.zeros_like(union))
        cond = (labels == li) & (iou > _NMS_THR) & (iota > i)

        keep_i = (supi == 0.0) & (si > 0.0)
        new_sup = jnp.where(
            keep_i, jnp.maximum(suppressed, cond.astype(jnp.float32)), suppressed
        )
        keep_val = jnp.where(keep_i, 1.0, 0.0)
        new_keep = jnp.where(onehot > 0.0, keep_val, keep)
        return new_sup, new_keep

    zeros = jnp.zeros((1, _KP), dtype=jnp.float32)
    _, keep = jax.lax.fori_loop(0, _K, body, (zeros, zeros))
    keep_ref[...] = keep


def kernel(cls_logits, bbox_regression, anchors):
    rows_pad = _NP - _N
    logits_p = jnp.pad(cls_logits, ((0, rows_pad), (0, 0)))
    reg_t = jnp.pad(bbox_regression.T, ((0, 0), (0, rows_pad)))
    anc_t = jnp.pad(anchors.T, ((0, 0), (0, rows_pad)))

    scores, boxes_t = pl.pallas_call(
        _prep_kernel,
        grid=(_GRID,),
        in_specs=[
            pl.BlockSpec((_BR, _C), lambda i: (i, 0)),
            pl.BlockSpec((4, _BR), lambda i: (0, i)),
            pl.BlockSpec((4, _BR), lambda i: (0, i)),
        ],
        out_specs=[
            pl.BlockSpec((_BR, _C - 1), lambda i: (i, 0)),
            pl.BlockSpec((4, _BR), lambda i: (0, i)),
        ],
        out_shape=[
            jax.ShapeDtypeStruct((_NP, _C - 1), jnp.float32),
            jax.ShapeDtypeStruct((4, _NP), jnp.float32),
        ],
    )(logits_p, reg_t, anc_t)
    scores = scores[:_N]
    boxes_t = boxes_t[:, :_N]

    num_fg = _C - 1
    flat_scores = scores.reshape(-1)
    top_scores, top_idx = jax.lax.top_k(flat_scores, _K)
    anchor_idx = top_idx // num_fg
    label_idx = top_idx % num_fg + 1

    top_boxes_t = jnp.take(boxes_t, anchor_idx, axis=1)

    pad = _KP - _K
    b_pad = jnp.pad(top_boxes_t, ((0, 0), (0, pad)))
    s_pad = jnp.pad(top_scores, (0, pad)).reshape(1, _KP)
    l_pad = jnp.pad(label_idx.astype(jnp.float32), (0, pad)).reshape(1, _KP)

    keep = pl.pallas_call(
        _nms_kernel,
        out_shape=jax.ShapeDtypeStruct((1, _KP), jnp.float32),
    )(b_pad, s_pad, l_pad)
    keep = keep[0, :_K]

    selected = jnp.where(keep > 0.0, top_scores, jnp.zeros_like(top_scores))
    final_scores, order = jax.lax.top_k(selected, _MAXD)
    final_boxes = jnp.take(top_boxes_t, order, axis=1).T
    final_labels = jnp.take(label_idx, order).astype(jnp.float32)
    valid = (final_scores > 0.0).astype(jnp.float32)
    final_scores = final_scores * valid
    final_labels = final_labels * valid
    scale = jnp.array([_W, _H, _W, _H], dtype=final_boxes.dtype)
    final_boxes = final_boxes / scale * valid[:, None]
    return jnp.concatenate(
        [final_labels[:, None], final_scores[:, None], final_boxes], axis=1
    )
